# baseline (speedup 1.0000x reference)
import functools

import jax
import jax.numpy as jnp
from jax import lax
from jax.experimental import pallas as pl
from jax.experimental.pallas import tpu as pltpu
from jax.experimental.pallas import tpu_sc as plsc

_B = 16384
_D = 64
_DP = 128
_MARGIN = 1.0
_NC = 2
_NS = 16
_L = 16
_NW = _NC * _NS
_BW = _B // _NW
_C = 128
_NCHUNK = _BW // _C
_NENT = 1000000
_NBLK = (_NENT + _DP - 1) // _DP
_BFULL = _NENT // _DP
_TPW = (_NBLK + _NW - 1) // _NW

_mesh = plsc.VectorSubcoreMesh(core_axis_name="c", subcore_axis_name="s")


@functools.partial(
    pl.kernel,
    mesh=_mesh,
    compiler_params=pltpu.CompilerParams(needs_layout_passes=False),
    out_type=jax.ShapeDtypeStruct((_NENT, _DP), jnp.float32),
    scratch_types=[
        pltpu.VMEM((_D, _DP), jnp.float32),
        pltpu.VMEM((_DP, _DP), jnp.float32),
    ],
)
def _format_sc(entT, tail, out, tbuf, obuf):
    wid = lax.axis_index("s") * _NC + lax.axis_index("c")
    lanes = lax.iota(jnp.int32, _L)

    def block(t, _):
        b = wid + t * _NW

        @pl.when(b < _BFULL)
        def _full():
            c0 = pl.multiple_of(b * _DP, _DP)
            pltpu.sync_copy(entT.at[:, pl.ds(c0, _DP)], tbuf)

            def row(l, _):
                for j in range(_D // _L):
                    col = plsc.load_gather(
                        tbuf, [lanes + j * _L, jnp.full((_L,), l, jnp.int32)])
                    obuf[l, pl.ds(j * _L, _L)] = col
                return 0

            lax.fori_loop(0, _DP, row, 0)
            pltpu.sync_copy(obuf, out.at[pl.ds(c0, _DP), :])

        @pl.when(b == _BFULL)
        def _narrow():
            pltpu.sync_copy(tail, tbuf)

            def row(l, _):
                for j in range(_D // _L):
                    col = plsc.load_gather(
                        tbuf, [lanes + j * _L, jnp.full((_L,), l, jnp.int32)])
                    obuf[l, pl.ds(j * _L, _L)] = col
                return 0

            lax.fori_loop(0, _DP, row, 0)
            pltpu.sync_copy(obuf, out.at[pl.ds(_NENT - _DP, _DP), :])

        return 0

    lax.fori_loop(0, _TPW, block, 0)


@functools.partial(
    pl.kernel,
    mesh=_mesh,
    compiler_params=pltpu.CompilerParams(needs_layout_passes=False),
    out_type=jax.ShapeDtypeStruct((_NW, _L), jnp.float32),
    scratch_types=[
        pltpu.VMEM((_C,), jnp.int32),
        pltpu.VMEM((_C,), jnp.int32),
        pltpu.VMEM((_C,), jnp.int32),
        pltpu.VMEM((_C,), jnp.int32),
        pltpu.VMEM((_C,), jnp.int32),
        pltpu.VMEM((_C,), jnp.int32),
        pltpu.VMEM((_C, _DP), jnp.float32),
        pltpu.VMEM((_C, _D), jnp.float32),
        pltpu.VMEM((_C, _DP), jnp.float32),
        pltpu.VMEM((_C, _DP), jnp.float32),
        pltpu.VMEM((_C, _D), jnp.float32),
        pltpu.VMEM((_C, _DP), jnp.float32),
        pltpu.VMEM((_L * _L,), jnp.float32),
        pltpu.VMEM((_L,), jnp.float32),
        pltpu.SemaphoreType.DMA,
        pltpu.SemaphoreType.DMA,
    ],
)
def _transe_sc(ph, pr, pt, nh, nr, nt, ent2, rel, out,
               iph, ipr, ipt, inh, inr, int_,
               rph, rpr, rpt, rnh, rnr, rnt, sbuf, accv, sem, rsem):
    wid = lax.axis_index("s") * _NC + lax.axis_index("c")
    base = wid * _BW
    lanes = lax.iota(jnp.int32, _L)

    def chunk(g, acc):
        cb = pl.multiple_of(base + g * _C, _C)
        pltpu.sync_copy(ph.at[pl.ds(cb, _C)], iph)
        pltpu.sync_copy(pr.at[pl.ds(cb, _C)], ipr)
        pltpu.sync_copy(pt.at[pl.ds(cb, _C)], ipt)
        pltpu.sync_copy(nh.at[pl.ds(cb, _C)], inh)
        pltpu.sync_copy(nr.at[pl.ds(cb, _C)], inr)
        pltpu.sync_copy(nt.at[pl.ds(cb, _C)], int_)

        cps = [
            pltpu.async_copy(ent2.at[iph], rph, sem),
            pltpu.async_copy(ent2.at[ipt], rpt, sem),
            pltpu.async_copy(ent2.at[inh], rnh, sem),
            pltpu.async_copy(ent2.at[int_], rnt, sem),
        ]

        def fire(k, _):
            i0 = k * _L
            vr = ipr[pl.ds(i0, _L)]
            wr = inr[pl.ds(i0, _L)]
            for m in range(_L):
                i = i0 + m
                pltpu.async_copy(rel.at[vr[m]], rpr.at[i], rsem)
                pltpu.async_copy(rel.at[wr[m]], rnr.at[i], rsem)
            return 0

        lax.fori_loop(0, _C // _L, fire, 0)
        for cp in cps:
            cp.wait()
        drain = pltpu.make_async_copy(rel.at[0], rpr.at[0], rsem)
        for _ in range(2 * _C):
            drain.wait()

        def group(k, a):
            i0 = k * _L
            for m in range(_L):
                i = i0 + m
                s = jnp.zeros((_L,), jnp.float32)
                for j in range(_D // _L):
                    d = pl.ds(j * _L, _L)
                    dp = jnp.abs(rph[i, d] + rpr[i, d] - rpt[i, d])
                    dn = jnp.abs(rnh[i, d] + rnr[i, d] - rnt[i, d])
                    s = s + (dp - dn)
                sbuf[pl.ds(m * _L, _L)] = s
            rs = jnp.zeros((_L,), jnp.float32)
            for d in range(_L):
                rs = rs + plsc.load_gather(sbuf, [lanes * _L + d])
            return a + jnp.maximum(0.0, _MARGIN + rs)

        return lax.fori_loop(0, _C // _L, group, acc)

    acc = lax.fori_loop(0, _NCHUNK, chunk, jnp.zeros((_L,), jnp.float32))
    accv[...] = acc
    pltpu.sync_copy(accv, out.at[wid])


def kernel(pos_h, pos_r, pos_t, neg_h, neg_r, neg_t, ent_emb, rel_emb):
    ent_t = ent_emb.T
    tail = lax.slice(ent_t, (0, _NENT - _DP), (_D, _NENT))
    ent2 = _format_sc(ent_t, tail)
    parts = _transe_sc(pos_h, pos_r, pos_t, neg_h, neg_r, neg_t,
                       ent2, rel_emb)
    return jnp.sum(parts) * (1.0 / _B)


# speedup vs baseline: 3.5314x; 3.5314x over previous
import functools

import jax
import jax.numpy as jnp
from jax import lax
from jax.experimental import pallas as pl
from jax.experimental.pallas import tpu as pltpu
from jax.experimental.pallas import tpu_sc as plsc

_B = 16384
_D = 64
_DP = 128
_MARGIN = 1.0
_NC = 2
_NS = 16
_L = 16
_NW = _NC * _NS
_BW = _B // _NW
_C = 128
_NCHUNK = _BW // _C
_NENT = 1000000
_NBLK = (_NENT + _DP - 1) // _DP
_BFULL = _NENT // _DP
_TPW = (_NBLK + _NW - 1) // _NW

_mesh = plsc.VectorSubcoreMesh(core_axis_name="c", subcore_axis_name="s")


@functools.partial(
    pl.kernel,
    mesh=_mesh,
    compiler_params=pltpu.CompilerParams(needs_layout_passes=False),
    out_type=jax.ShapeDtypeStruct((_NENT, _DP), jnp.float32),
    scratch_types=[
        pltpu.VMEM((_D, _DP), jnp.float32),
        pltpu.VMEM((_D, _DP), jnp.float32),
        pltpu.VMEM((_DP, _DP), jnp.float32),
        pltpu.VMEM((_DP, _DP), jnp.float32),
        pltpu.SemaphoreType.DMA,
        pltpu.SemaphoreType.DMA,
        pltpu.SemaphoreType.DMA,
        pltpu.SemaphoreType.DMA,
    ],
)
def _format_sc(entT, tail, out, tb0, tb1, ob0, ob1, is0, is1, os0, os1):
    wid = lax.axis_index("s") * _NC + lax.axis_index("c")
    lanes = lax.iota(jnp.int32, _L)
    tpw = (_NBLK - wid + _NW - 1) // _NW

    def start_in(t, tb, isem):
        b = wid + t * _NW

        @pl.when(b < _BFULL)
        def _():
            c0 = pl.multiple_of(b * _DP, _DP)
            pltpu.async_copy(entT.at[:, pl.ds(c0, _DP)], tb, isem)

        @pl.when(b == _BFULL)
        def _():
            pltpu.async_copy(tail, tb, isem)

    def transpose_block(tb, ob):
        def rotation(c, _):
            rot = (lanes + c) & (_L - 1)
            for j in range(_D // _L):
                rows = rot + j * _L
                for k in range(_DP // _L):
                    cols = lanes + k * _L
                    v = plsc.load_gather(tb, [rows, cols])
                    plsc.store_scatter(ob, [cols, rows], v)
            return 0

        lax.fori_loop(0, _L, rotation, 0)

    def finish_block(t, tb, ob, isem, osem):
        b = wid + t * _NW
        pltpu.make_async_copy(entT.at[:, pl.ds(0, _DP)], tb, isem).wait()
        transpose_block(tb, ob)
        c0 = jnp.minimum(b * _DP, _NENT - _DP)
        pltpu.async_copy(ob, out.at[pl.ds(pl.multiple_of(c0, 8), _DP), :],
                         osem)

    start_in(0, tb0, is0)

    def pair(p, _):
        t0 = 2 * p
        t1 = 2 * p + 1

        @pl.when(t0 < tpw)
        def _():
            @pl.when(t1 < tpw)
            def _():
                start_in(t1, tb1, is1)

            @pl.when(t0 >= 2)
            def _():
                pltpu.make_async_copy(
                    ob0, out.at[pl.ds(0, _DP), :], os0).wait()
            finish_block(t0, tb0, ob0, is0, os0)

        @pl.when(t1 < tpw)
        def _():
            @pl.when(t1 + 1 < tpw)
            def _():
                start_in(t1 + 1, tb0, is0)

            @pl.when(t1 >= 2)
            def _():
                pltpu.make_async_copy(
                    ob1, out.at[pl.ds(0, _DP), :], os1).wait()
            finish_block(t1, tb1, ob1, is1, os1)

        return 0

    lax.fori_loop(0, (_TPW + 1) // 2, pair, 0)
    pltpu.make_async_copy(ob0, out.at[pl.ds(0, _DP), :], os0).wait()
    pltpu.make_async_copy(ob1, out.at[pl.ds(0, _DP), :], os1).wait()


@functools.partial(
    pl.kernel,
    mesh=_mesh,
    compiler_params=pltpu.CompilerParams(needs_layout_passes=False),
    out_type=jax.ShapeDtypeStruct((_NW, _L), jnp.float32),
    scratch_types=[
        pltpu.VMEM((_C,), jnp.int32),
        pltpu.VMEM((_C,), jnp.int32),
        pltpu.VMEM((_C,), jnp.int32),
        pltpu.VMEM((_C,), jnp.int32),
        pltpu.VMEM((_C,), jnp.int32),
        pltpu.VMEM((_C,), jnp.int32),
        pltpu.VMEM((_C, _DP), jnp.float32),
        pltpu.VMEM((_C, _D), jnp.float32),
        pltpu.VMEM((_C, _DP), jnp.float32),
        pltpu.VMEM((_C, _DP), jnp.float32),
        pltpu.VMEM((_C, _D), jnp.float32),
        pltpu.VMEM((_C, _DP), jnp.float32),
        pltpu.VMEM((_L * _L,), jnp.float32),
        pltpu.VMEM((_L,), jnp.float32),
        pltpu.SemaphoreType.DMA,
        pltpu.SemaphoreType.DMA,
    ],
)
def _transe_sc(ph, pr, pt, nh, nr, nt, ent2, rel, out,
               iph, ipr, ipt, inh, inr, int_,
               rph, rpr, rpt, rnh, rnr, rnt, sbuf, accv, sem, rsem):
    wid = lax.axis_index("s") * _NC + lax.axis_index("c")
    base = wid * _BW
    lanes = lax.iota(jnp.int32, _L)

    def chunk(g, acc):
        cb = pl.multiple_of(base + g * _C, _C)
        pltpu.sync_copy(ph.at[pl.ds(cb, _C)], iph)
        pltpu.sync_copy(pr.at[pl.ds(cb, _C)], ipr)
        pltpu.sync_copy(pt.at[pl.ds(cb, _C)], ipt)
        pltpu.sync_copy(nh.at[pl.ds(cb, _C)], inh)
        pltpu.sync_copy(nr.at[pl.ds(cb, _C)], inr)
        pltpu.sync_copy(nt.at[pl.ds(cb, _C)], int_)

        cps = [
            pltpu.async_copy(ent2.at[iph], rph, sem),
            pltpu.async_copy(ent2.at[ipt], rpt, sem),
            pltpu.async_copy(ent2.at[inh], rnh, sem),
            pltpu.async_copy(ent2.at[int_], rnt, sem),
        ]

        def fire(k, _):
            i0 = k * _L
            vr = ipr[pl.ds(i0, _L)]
            wr = inr[pl.ds(i0, _L)]
            for m in range(_L):
                i = i0 + m
                pltpu.async_copy(rel.at[vr[m]], rpr.at[i], rsem)
                pltpu.async_copy(rel.at[wr[m]], rnr.at[i], rsem)
            return 0

        lax.fori_loop(0, _C // _L, fire, 0)
        for cp in cps:
            cp.wait()
        drain = pltpu.make_async_copy(rel.at[0], rpr.at[0], rsem)
        for _ in range(2 * _C):
            drain.wait()

        def group(k, a):
            i0 = k * _L
            for m in range(_L):
                i = i0 + m
                s = jnp.zeros((_L,), jnp.float32)
                for j in range(_D // _L):
                    d = pl.ds(j * _L, _L)
                    dp = jnp.abs(rph[i, d] + rpr[i, d] - rpt[i, d])
                    dn = jnp.abs(rnh[i, d] + rnr[i, d] - rnt[i, d])
                    s = s + (dp - dn)
                sbuf[pl.ds(m * _L, _L)] = s
            rs = jnp.zeros((_L,), jnp.float32)
            for d in range(_L):
                rs = rs + plsc.load_gather(sbuf, [lanes * _L + d])
            return a + jnp.maximum(0.0, _MARGIN + rs)

        return lax.fori_loop(0, _C // _L, group, acc)

    acc = lax.fori_loop(0, _NCHUNK, chunk, jnp.zeros((_L,), jnp.float32))
    accv[...] = acc
    pltpu.sync_copy(accv, out.at[wid])


def kernel(pos_h, pos_r, pos_t, neg_h, neg_r, neg_t, ent_emb, rel_emb):
    ent_t = ent_emb.T
    tail = lax.slice(ent_t, (0, _NENT - _DP), (_D, _NENT))
    ent2 = _format_sc(ent_t, tail)
    parts = _transe_sc(pos_h, pos_r, pos_t, neg_h, neg_r, neg_t,
                       ent2, rel_emb)
    return jnp.sum(parts) * (1.0 / _B)


# speedup vs baseline: 3.6537x; 1.0346x over previous
import functools

import jax
import jax.numpy as jnp
from jax import lax
from jax.experimental import pallas as pl
from jax.experimental.pallas import tpu as pltpu
from jax.experimental.pallas import tpu_sc as plsc

_B = 16384
_D = 64
_DP = 128
_MARGIN = 1.0
_NC = 2
_NS = 16
_L = 16
_NW = _NC * _NS
_BW = _B // _NW
_C = 128
_NCHUNK = _BW // _C
_NENT = 1000000
_NBLK = (_NENT + _DP - 1) // _DP
_BFULL = _NENT // _DP
_TPW = (_NBLK + _NW - 1) // _NW

_mesh = plsc.VectorSubcoreMesh(core_axis_name="c", subcore_axis_name="s")


@functools.partial(
    pl.kernel,
    mesh=_mesh,
    compiler_params=pltpu.CompilerParams(needs_layout_passes=False),
    out_type=jax.ShapeDtypeStruct((_NENT, _DP), jnp.float32),
    scratch_types=[
        pltpu.VMEM((_D, _DP), jnp.float32),
        pltpu.VMEM((_D, _DP), jnp.float32),
        pltpu.VMEM((_DP, _DP), jnp.float32),
        pltpu.VMEM((_DP, _DP), jnp.float32),
        pltpu.SemaphoreType.DMA,
        pltpu.SemaphoreType.DMA,
        pltpu.SemaphoreType.DMA,
        pltpu.SemaphoreType.DMA,
    ],
)
def _format_sc(entT, tail, out, tb0, tb1, ob0, ob1, is0, is1, os0, os1):
    wid = lax.axis_index("s") * _NC + lax.axis_index("c")
    lanes = lax.iota(jnp.int32, _L)
    tpw = (_NBLK - wid + _NW - 1) // _NW

    def start_in(t, tb, isem):
        b = wid + t * _NW

        @pl.when(b < _BFULL)
        def _():
            c0 = pl.multiple_of(b * _DP, _DP)
            pltpu.async_copy(entT.at[:, pl.ds(c0, _DP)], tb, isem)

        @pl.when(b == _BFULL)
        def _():
            pltpu.async_copy(tail, tb, isem)

    def transpose_block(tb, ob):
        def rotation(c, _):
            rot = (lanes + c) & (_L - 1)
            for j in range(_D // _L):
                rows = rot + j * _L
                for k in range(_DP // _L):
                    cols = lanes + k * _L
                    v = plsc.load_gather(tb, [rows, cols])
                    plsc.store_scatter(ob, [cols, rows], v)
            return 0

        lax.fori_loop(0, _L, rotation, 0, unroll=4)

    def finish_block(t, tb, ob, isem, osem):
        b = wid + t * _NW
        pltpu.make_async_copy(entT.at[:, pl.ds(0, _DP)], tb, isem).wait()
        transpose_block(tb, ob)
        c0 = jnp.minimum(b * _DP, _NENT - _DP)
        pltpu.async_copy(ob, out.at[pl.ds(pl.multiple_of(c0, 8), _DP), :],
                         osem)

    start_in(0, tb0, is0)

    def pair(p, _):
        t0 = 2 * p
        t1 = 2 * p + 1

        @pl.when(t0 < tpw)
        def _():
            @pl.when(t1 < tpw)
            def _():
                start_in(t1, tb1, is1)

            @pl.when(t0 >= 2)
            def _():
                pltpu.make_async_copy(
                    ob0, out.at[pl.ds(0, _DP), :], os0).wait()
            finish_block(t0, tb0, ob0, is0, os0)

        @pl.when(t1 < tpw)
        def _():
            @pl.when(t1 + 1 < tpw)
            def _():
                start_in(t1 + 1, tb0, is0)

            @pl.when(t1 >= 2)
            def _():
                pltpu.make_async_copy(
                    ob1, out.at[pl.ds(0, _DP), :], os1).wait()
            finish_block(t1, tb1, ob1, is1, os1)

        return 0

    lax.fori_loop(0, (_TPW + 1) // 2, pair, 0)
    pltpu.make_async_copy(ob0, out.at[pl.ds(0, _DP), :], os0).wait()
    pltpu.make_async_copy(ob1, out.at[pl.ds(0, _DP), :], os1).wait()


@functools.partial(
    pl.kernel,
    mesh=_mesh,
    compiler_params=pltpu.CompilerParams(needs_layout_passes=False),
    out_type=jax.ShapeDtypeStruct((_NW, _L), jnp.float32),
    scratch_types=[
        pltpu.VMEM((_C,), jnp.int32),
        pltpu.VMEM((_C,), jnp.int32),
        pltpu.VMEM((_C,), jnp.int32),
        pltpu.VMEM((_C,), jnp.int32),
        pltpu.VMEM((_C,), jnp.int32),
        pltpu.VMEM((_C,), jnp.int32),
        pltpu.VMEM((_C, _DP), jnp.float32),
        pltpu.VMEM((_C, _D), jnp.float32),
        pltpu.VMEM((_C, _DP), jnp.float32),
        pltpu.VMEM((_C, _DP), jnp.float32),
        pltpu.VMEM((_C, _D), jnp.float32),
        pltpu.VMEM((_C, _DP), jnp.float32),
        pltpu.VMEM((_L * _L,), jnp.float32),
        pltpu.VMEM((_L,), jnp.float32),
        pltpu.SemaphoreType.DMA,
        pltpu.SemaphoreType.DMA,
    ],
)
def _transe_sc(ph, pr, pt, nh, nr, nt, ent2, rel, out,
               iph, ipr, ipt, inh, inr, int_,
               rph, rpr, rpt, rnh, rnr, rnt, sbuf, accv, sem, rsem):
    wid = lax.axis_index("s") * _NC + lax.axis_index("c")
    base = wid * _BW
    lanes = lax.iota(jnp.int32, _L)

    def chunk(g, acc):
        cb = pl.multiple_of(base + g * _C, _C)
        pltpu.sync_copy(ph.at[pl.ds(cb, _C)], iph)
        pltpu.sync_copy(pr.at[pl.ds(cb, _C)], ipr)
        pltpu.sync_copy(pt.at[pl.ds(cb, _C)], ipt)
        pltpu.sync_copy(nh.at[pl.ds(cb, _C)], inh)
        pltpu.sync_copy(nr.at[pl.ds(cb, _C)], inr)
        pltpu.sync_copy(nt.at[pl.ds(cb, _C)], int_)

        cps = [
            pltpu.async_copy(ent2.at[iph], rph, sem),
            pltpu.async_copy(ent2.at[ipt], rpt, sem),
            pltpu.async_copy(ent2.at[inh], rnh, sem),
            pltpu.async_copy(ent2.at[int_], rnt, sem),
        ]

        def fire(k, _):
            i0 = k * _L
            vr = ipr[pl.ds(i0, _L)]
            wr = inr[pl.ds(i0, _L)]
            for m in range(_L):
                i = i0 + m
                pltpu.async_copy(rel.at[vr[m]], rpr.at[i], rsem)
                pltpu.async_copy(rel.at[wr[m]], rnr.at[i], rsem)
            return 0

        lax.fori_loop(0, _C // _L, fire, 0)
        for cp in cps:
            cp.wait()
        drain = pltpu.make_async_copy(rel.at[0], rpr.at[0], rsem)
        for _ in range(2 * _C):
            drain.wait()

        def group(k, a):
            i0 = k * _L
            for m in range(_L):
                i = i0 + m
                s = jnp.zeros((_L,), jnp.float32)
                for j in range(_D // _L):
                    d = pl.ds(j * _L, _L)
                    dp = jnp.abs(rph[i, d] + rpr[i, d] - rpt[i, d])
                    dn = jnp.abs(rnh[i, d] + rnr[i, d] - rnt[i, d])
                    s = s + (dp - dn)
                sbuf[pl.ds(m * _L, _L)] = s
            rs = jnp.zeros((_L,), jnp.float32)
            for d in range(_L):
                rs = rs + plsc.load_gather(sbuf, [lanes * _L + d])
            return a + jnp.maximum(0.0, _MARGIN + rs)

        return lax.fori_loop(0, _C // _L, group, acc)

    acc = lax.fori_loop(0, _NCHUNK, chunk, jnp.zeros((_L,), jnp.float32))
    accv[...] = acc
    pltpu.sync_copy(accv, out.at[wid])


def kernel(pos_h, pos_r, pos_t, neg_h, neg_r, neg_t, ent_emb, rel_emb):
    ent_t = ent_emb.T
    tail = lax.slice(ent_t, (0, _NENT - _DP), (_D, _NENT))
    ent2 = _format_sc(ent_t, tail)
    parts = _transe_sc(pos_h, pos_r, pos_t, neg_h, neg_r, neg_t,
                       ent2, rel_emb)
    return jnp.sum(parts) * (1.0 / _B)


# speedup vs baseline: 4.8595x; 1.3300x over previous
import functools

import jax
import jax.numpy as jnp
from jax import lax
from jax.experimental import pallas as pl
from jax.experimental.pallas import tpu as pltpu
from jax.experimental.pallas import tpu_sc as plsc

_B = 16384
_D = 64
_MARGIN = 1.0
_NC = 2
_NS = 16
_L = 16
_NW = _NC * _NS
_BW = _B // _NW
_C = 128
_NCHUNK = _BW // _C

_mesh = plsc.VectorSubcoreMesh(core_axis_name="c", subcore_axis_name="s")


@functools.partial(
    pl.kernel,
    mesh=_mesh,
    compiler_params=pltpu.CompilerParams(needs_layout_passes=False),
    out_type=jax.ShapeDtypeStruct((_NW, _L), jnp.float32),
    scratch_types=[
        pltpu.VMEM((_C,), jnp.int32),
        pltpu.VMEM((_C,), jnp.int32),
        pltpu.VMEM((_C,), jnp.int32),
        pltpu.VMEM((_C,), jnp.int32),
        pltpu.VMEM((_C,), jnp.int32),
        pltpu.VMEM((_C,), jnp.int32),
        pltpu.VMEM((_C, _D), jnp.float32),
        pltpu.VMEM((_C, _D), jnp.float32),
        pltpu.VMEM((_C, _D), jnp.float32),
        pltpu.VMEM((_C, _D), jnp.float32),
        pltpu.VMEM((_C, _D), jnp.float32),
        pltpu.VMEM((_C, _D), jnp.float32),
        pltpu.VMEM((_L * _L,), jnp.float32),
        pltpu.VMEM((_L,), jnp.float32),
        pltpu.SemaphoreType.DMA,
    ],
)
def _transe_sc(ph, pr, pt, nh, nr, nt, ent, rel, out,
               iph, ipr, ipt, inh, inr, int_,
               rph, rpr, rpt, rnh, rnr, rnt, sbuf, accv, sem):
    wid = lax.axis_index("s") * _NC + lax.axis_index("c")
    base = wid * _BW
    lanes = lax.iota(jnp.int32, _L)

    def chunk(g, acc):
        cb = pl.multiple_of(base + g * _C, _C)
        pltpu.sync_copy(ph.at[pl.ds(cb, _C)], iph)
        pltpu.sync_copy(pr.at[pl.ds(cb, _C)], ipr)
        pltpu.sync_copy(pt.at[pl.ds(cb, _C)], ipt)
        pltpu.sync_copy(nh.at[pl.ds(cb, _C)], inh)
        pltpu.sync_copy(nr.at[pl.ds(cb, _C)], inr)
        pltpu.sync_copy(nt.at[pl.ds(cb, _C)], int_)

        def fire(k, _):
            i0 = k * _L
            vh = iph[pl.ds(i0, _L)]
            vr = ipr[pl.ds(i0, _L)]
            vt = ipt[pl.ds(i0, _L)]
            wh = inh[pl.ds(i0, _L)]
            wr = inr[pl.ds(i0, _L)]
            wt = int_[pl.ds(i0, _L)]
            for m in range(_L):
                i = i0 + m
                pltpu.async_copy(ent.at[vh[m]], rph.at[i], sem)
                pltpu.async_copy(rel.at[vr[m]], rpr.at[i], sem)
                pltpu.async_copy(ent.at[vt[m]], rpt.at[i], sem)
                pltpu.async_copy(ent.at[wh[m]], rnh.at[i], sem)
                pltpu.async_copy(rel.at[wr[m]], rnr.at[i], sem)
                pltpu.async_copy(ent.at[wt[m]], rnt.at[i], sem)
            return 0

        lax.fori_loop(0, _C // _L, fire, 0)
        drain = pltpu.make_async_copy(ent.at[0], rph.at[0], sem)
        for _ in range(6 * _C):
            drain.wait()

        def group(k, a):
            i0 = k * _L
            for m in range(_L):
                i = i0 + m
                s = jnp.zeros((_L,), jnp.float32)
                for j in range(_D // _L):
                    d = pl.ds(j * _L, _L)
                    dp = jnp.abs(rph[i, d] + rpr[i, d] - rpt[i, d])
                    dn = jnp.abs(rnh[i, d] + rnr[i, d] - rnt[i, d])
                    s = s + (dp - dn)
                sbuf[pl.ds(m * _L, _L)] = s
            rs = jnp.zeros((_L,), jnp.float32)
            for d in range(_L):
                rs = rs + plsc.load_gather(sbuf, [lanes * _L + d])
            return a + jnp.maximum(0.0, _MARGIN + rs)

        return lax.fori_loop(0, _C // _L, group, acc)

    acc = lax.fori_loop(0, _NCHUNK, chunk, jnp.zeros((_L,), jnp.float32))
    accv[...] = acc
    pltpu.sync_copy(accv, out.at[wid])


def kernel(pos_h, pos_r, pos_t, neg_h, neg_r, neg_t, ent_emb, rel_emb):
    parts = _transe_sc(pos_h, pos_r, pos_t, neg_h, neg_r, neg_t,
                       ent_emb, rel_emb)
    return jnp.sum(parts) * (1.0 / _B)
